# baseline (speedup 1.0000x reference)
import functools

import jax
import jax.numpy as jnp
from jax import lax
from jax.experimental import pallas as pl
from jax.experimental.pallas import tpu as pltpu
from jax.experimental.pallas import tpu_sc as plsc

BS = 2
NQ = 2048
NV = 2048
D = 256
H = 8
P = 32
HD = D // H

BQ = 512
NC, NS = 2, 16
NW = NC * NS
QPW = (BS * NQ * H) // (NW * H) * 1
QC = 256
LG = 16
DB = 32
UNROLL = 4


def _pack_pair_words(lo, hi):
    lob = lo.astype(jnp.bfloat16).astype(jnp.float32)
    hib = hi.astype(jnp.bfloat16).astype(jnp.float32)
    lou = lax.bitcast_convert_type(lob, jnp.uint32) >> 16
    hiu = lax.bitcast_convert_type(hib, jnp.uint32) & jnp.uint32(0xFFFF0000)
    return lax.bitcast_convert_type(lou | hiu, jnp.int32)


def _stage1_body(q_ref, v_ref, r_ref, wv_ref, bv_ref, wo_ref, bo_ref,
                 wsw_ref, bsw_ref, vout_ref, c_ref, j_ref):
    q = q_ref[0]
    v = v_ref[0]
    vp = jnp.dot(v, wv_ref[...].T, preferred_element_type=jnp.float32) + bv_ref[...]
    vout_ref[0] = vp.T.reshape(H, HD, BQ)

    off = jnp.dot(q, wo_ref[...].T, preferred_element_type=jnp.float32) + bo_ref[...]
    r = r_ref[0]
    x = r * float(NV) - 1.0 + off

    sw = jnp.dot(q, wsw_ref[...].T, preferred_element_type=jnp.float32) + bsw_ref[...]
    sw3 = sw.reshape(BQ, H, P)
    m = jnp.max(sw3, axis=-1, keepdims=True)
    e = jnp.exp(sw3 - m)
    s = jnp.sum(e, axis=-1, keepdims=True)
    w = (e / s).reshape(BQ, H * P)

    xl = jnp.floor(x)
    lx = x - xl
    hx = 1.0 - lx
    a = jnp.where((xl >= 0.0) & (xl <= float(NV - 2)), w * hx, 0.0)
    bb = (jnp.where((xl >= -1.0) & (xl <= float(NV - 2)), w * lx, 0.0)
          + jnp.where(xl == float(NV - 1), w * hx, 0.0))
    jv = jnp.clip(xl + 1.0, 0.0, float(NV - 1)).astype(jnp.int32)
    cw = _pack_pair_words(a, bb)
    c_ref[0] = cw.T.reshape(H, P, BQ)
    j_ref[0] = jv.T.reshape(H, P, BQ)


def _stage1(query, value, ref_pts, W_v, b_v, W_off, b_off, W_sw, b_sw):
    grid = (BS, NQ // BQ)
    blk_rows = pl.BlockSpec((1, BQ, D), lambda b, i: (b, i, 0))
    blk_full = pl.BlockSpec((D, D), lambda b, i: (0, 0))
    blk_vec = pl.BlockSpec((D,), lambda b, i: (0,))
    blk_r = pl.BlockSpec((1, BQ, 1), lambda b, i: (b, i, 0))
    blk_t = pl.BlockSpec((1, H, P, BQ), lambda b, i: (b, 0, 0, i))
    blk_vt = pl.BlockSpec((1, H, HD, BQ), lambda b, i: (b, 0, 0, i))
    return pl.pallas_call(
        _stage1_body,
        grid=grid,
        in_specs=[blk_rows, blk_rows, blk_r, blk_full, blk_vec, blk_full,
                  blk_vec, blk_full, blk_vec],
        out_specs=[blk_vt, blk_t, blk_t],
        out_shape=[
            jax.ShapeDtypeStruct((BS, H, HD, NV), jnp.float32),
            jax.ShapeDtypeStruct((BS, H, P, NQ), jnp.int32),
            jax.ShapeDtypeStruct((BS, H, P, NQ), jnp.int32),
        ],
    )(query, value, ref_pts, W_v, b_v, W_off, b_off, W_sw, b_sw)


def _pack_body(v_ref, t_ref):
    v = v_ref[0, 0]
    lo = jnp.concatenate([v[:, :1], v[:, :NV - 1]], axis=1)
    t_ref[0, 0] = _pack_pair_words(lo, v)


def _pack_table(v_proj):
    return pl.pallas_call(
        _pack_body,
        grid=(BS, H),
        in_specs=[pl.BlockSpec((1, 1, HD, NV), lambda b, h: (b, h, 0, 0))],
        out_specs=pl.BlockSpec((1, 1, HD, PSTRIDE), lambda b, h: (b, h, 0, 0)),
        out_shape=jax.ShapeDtypeStruct((BS, H, HD, PSTRIDE), jnp.int32),
    )(v_proj)


PSTRIDE = NV


def _sc_attend(pair_tab, cpk, jvi):
    mesh = plsc.VectorSubcoreMesh(core_axis_name="c", subcore_axis_name="s",
                                  num_cores=NC, num_subcores=NS)

    @functools.partial(
        pl.kernel,
        out_type=jax.ShapeDtypeStruct((BS, H, HD, NQ), jnp.float32),
        mesh=mesh,
        compiler_params=pltpu.CompilerParams(needs_layout_passes=False),
        scratch_types=[
            pltpu.VMEM((HD * PSTRIDE,), jnp.int32),
            pltpu.VMEM((P, QC), jnp.int32),
            pltpu.VMEM((P, QC), jnp.int32),
            pltpu.VMEM((HD, QC), jnp.float32),
        ],
    )
    def sc_kernel(t_hbm, c_hbm, j_hbm, out_hbm, ptab, cbuf, jbuf, obuf):
        cid = lax.axis_index("c")
        sid = lax.axis_index("s")
        wid = sid * NC + cid
        bh = wid % (BS * H)
        half = wid // (BS * H)
        b = bh // H
        h = bh % H

        pltpu.sync_copy(t_hbm.at[b, h], ptab)

        def do_chunk(qs):
            pltpu.sync_copy(c_hbm.at[b, h, :, pl.ds(qs, QC)], cbuf)
            pltpu.sync_copy(j_hbm.at[b, h, :, pl.ds(qs, QC)], jbuf)

            def group_body(g, _):
                qoff = g * LG

                init = tuple(jnp.zeros((2 * LG,), jnp.bfloat16)
                             for _ in range(DB))

                @plsc.parallel_loop(0, P, carry=init, unroll=UNROLL)
                def accs(p, accs):
                    jv = jbuf[p, pl.ds(qoff, LG)]
                    cv = plsc.bitcast(cbuf[p, pl.ds(qoff, LG)], jnp.bfloat16)
                    new = []
                    for j in range(DB):
                        tab = ptab.at[pl.ds(j * PSTRIDE, PSTRIDE)]
                        gv = plsc.load_gather(tab, [jv])
                        gb = plsc.bitcast(gv, jnp.bfloat16)
                        new.append(accs[j] + gb * cv)
                    return tuple(new)

                for j in range(DB):
                    lo, hi = plsc.unpack(accs[j],
                                         format=plsc.PackFormat.INTERLEAVED)
                    obuf[j, pl.ds(qoff, LG)] = lo + hi
                return 0

            lax.fori_loop(0, QC // LG, group_body, 0)
            pltpu.sync_copy(obuf, out_hbm.at[b, h, :, pl.ds(qs, QC)])

        q0 = half * (NQ // 2)
        for c in range(NQ // 2 // QC):
            do_chunk(q0 + c * QC)

    return sc_kernel(pair_tab, cpk, jvi)


def _stage3_body(ao_ref, q_ref, wo_ref, bo_ref, g_ref, beta_ref, out_ref):
    a = ao_ref[0].reshape(D, BQ).T
    o = jnp.dot(a, wo_ref[...].T, preferred_element_type=jnp.float32)
    o = o + bo_ref[...] + q_ref[0]
    mu = jnp.mean(o, axis=-1, keepdims=True)
    var = jnp.mean((o - mu) ** 2, axis=-1, keepdims=True)
    out_ref[0] = (o - mu) / jnp.sqrt(var + 1e-5) * g_ref[...] + beta_ref[...]


def _stage3(attn_t, query, W_out, b_out, ln_g, ln_b):
    grid = (BS, NQ // BQ)
    blk_rows = pl.BlockSpec((1, BQ, D), lambda b, i: (b, i, 0))
    blk_a = pl.BlockSpec((1, H, HD, BQ), lambda b, i: (b, 0, 0, i))
    blk_full = pl.BlockSpec((D, D), lambda b, i: (0, 0))
    blk_vec = pl.BlockSpec((D,), lambda b, i: (0,))
    return pl.pallas_call(
        _stage3_body,
        grid=grid,
        in_specs=[blk_a, blk_rows, blk_full, blk_vec, blk_vec, blk_vec],
        out_specs=blk_rows,
        out_shape=jax.ShapeDtypeStruct((BS, NQ, D), jnp.float32),
    )(attn_t, query, W_out, b_out, ln_g, ln_b)


@jax.jit
def _run(query, value, reference_points, W_off, b_off, W_sw, b_sw, W_v, b_v,
         W_out, b_out, ln_g, ln_b, spatial_shapes):
    ref2 = reference_points.reshape(BS, NQ, 1)
    v_proj, cpk, jvi = _stage1(query, value, ref2, W_v, b_v,
                               W_off, b_off, W_sw, b_sw)
    pair_tab = _pack_table(v_proj).reshape(BS, H, HD * PSTRIDE)
    attn_t = _sc_attend(pair_tab, cpk, jvi)
    return _stage3(attn_t, query, W_out, b_out, ln_g, ln_b)


def kernel(query, value, reference_points, W_off, b_off, W_sw, b_sw, W_v, b_v,
           W_out, b_out, ln_g, ln_b, spatial_shapes):
    return _run(query, value, reference_points, W_off, b_off, W_sw, b_sw,
                W_v, b_v, W_out, b_out, ln_g, ln_b, spatial_shapes)


# speedup vs baseline: 1.1605x; 1.1605x over previous
import functools

import jax
import jax.numpy as jnp
from jax import lax
from jax.experimental import pallas as pl
from jax.experimental.pallas import tpu as pltpu
from jax.experimental.pallas import tpu_sc as plsc

BS = 2
NQ = 2048
NV = 2048
D = 256
H = 8
P = 32
HD = D // H

BQ = 512
NC, NS = 2, 16
NW = NC * NS
QPW = (BS * NQ * H) // (NW * H) * 1
QC = 256
LG = 16
DB = 32
UNROLL = 2


def _pack_pair_words(lo, hi):
    lob = lo.astype(jnp.bfloat16).astype(jnp.float32)
    hib = hi.astype(jnp.bfloat16).astype(jnp.float32)
    lou = lax.bitcast_convert_type(lob, jnp.uint32) >> 16
    hiu = lax.bitcast_convert_type(hib, jnp.uint32) & jnp.uint32(0xFFFF0000)
    return lax.bitcast_convert_type(lou | hiu, jnp.int32)


def _stage1_body(q_ref, v_ref, r_ref, wv_ref, bv_ref, wo_ref, bo_ref,
                 wsw_ref, bsw_ref, vout_ref, c_ref, j_ref):
    q = q_ref[0]
    v = v_ref[0]
    vp = jnp.dot(v, wv_ref[...].T, preferred_element_type=jnp.float32) + bv_ref[...]
    vout_ref[0] = vp.T.reshape(H, HD, BQ)

    off = jnp.dot(q, wo_ref[...].T, preferred_element_type=jnp.float32) + bo_ref[...]
    r = r_ref[0]
    x = r * float(NV) - 1.0 + off

    sw = jnp.dot(q, wsw_ref[...].T, preferred_element_type=jnp.float32) + bsw_ref[...]
    sw3 = sw.reshape(BQ, H, P)
    m = jnp.max(sw3, axis=-1, keepdims=True)
    e = jnp.exp(sw3 - m)
    s = jnp.sum(e, axis=-1, keepdims=True)
    w = (e / s).reshape(BQ, H * P)

    xl = jnp.floor(x)
    lx = x - xl
    hx = 1.0 - lx
    a = jnp.where((xl >= 0.0) & (xl <= float(NV - 2)), w * hx, 0.0)
    bb = (jnp.where((xl >= -1.0) & (xl <= float(NV - 2)), w * lx, 0.0)
          + jnp.where(xl == float(NV - 1), w * hx, 0.0))
    jv = jnp.clip(xl + 1.0, 0.0, float(NV - 1)).astype(jnp.int32)
    cw = _pack_pair_words(a, bb)
    c_ref[0] = cw.T.reshape(H, P, BQ)
    j_ref[0] = jv.T.reshape(H, P, BQ)


def _stage1(query, value, ref_pts, W_v, b_v, W_off, b_off, W_sw, b_sw):
    grid = (BS, NQ // BQ)
    blk_rows = pl.BlockSpec((1, BQ, D), lambda b, i: (b, i, 0))
    blk_full = pl.BlockSpec((D, D), lambda b, i: (0, 0))
    blk_vec = pl.BlockSpec((D,), lambda b, i: (0,))
    blk_r = pl.BlockSpec((1, BQ, 1), lambda b, i: (b, i, 0))
    blk_t = pl.BlockSpec((1, H, P, BQ), lambda b, i: (b, 0, 0, i))
    blk_vt = pl.BlockSpec((1, H, HD, BQ), lambda b, i: (b, 0, 0, i))
    return pl.pallas_call(
        _stage1_body,
        grid=grid,
        in_specs=[blk_rows, blk_rows, blk_r, blk_full, blk_vec, blk_full,
                  blk_vec, blk_full, blk_vec],
        out_specs=[blk_vt, blk_t, blk_t],
        out_shape=[
            jax.ShapeDtypeStruct((BS, H, HD, NV), jnp.float32),
            jax.ShapeDtypeStruct((BS, H, P, NQ), jnp.int32),
            jax.ShapeDtypeStruct((BS, H, P, NQ), jnp.int32),
        ],
    )(query, value, ref_pts, W_v, b_v, W_off, b_off, W_sw, b_sw)


def _pack_body(v_ref, t_ref):
    v = v_ref[0, 0]
    z = jnp.zeros((HD, PSTRIDE - NV), jnp.float32)
    lo = jnp.concatenate([v[:, :1], v[:, :NV - 1], z], axis=1)
    hi = jnp.concatenate([v, z], axis=1)
    t_ref[0, 0] = _pack_pair_words(lo, hi)


def _pack_table(v_proj):
    return pl.pallas_call(
        _pack_body,
        grid=(BS, H),
        in_specs=[pl.BlockSpec((1, 1, HD, NV), lambda b, h: (b, h, 0, 0))],
        out_specs=pl.BlockSpec((1, 1, HD, PSTRIDE), lambda b, h: (b, h, 0, 0)),
        out_shape=jax.ShapeDtypeStruct((BS, H, HD, PSTRIDE), jnp.int32),
    )(v_proj)


PSTRIDE = 2064


def _sc_attend(pair_tab, cpk, jvi):
    mesh = plsc.VectorSubcoreMesh(core_axis_name="c", subcore_axis_name="s",
                                  num_cores=NC, num_subcores=NS)

    @functools.partial(
        pl.kernel,
        out_type=jax.ShapeDtypeStruct((BS, H, HD, NQ), jnp.float32),
        mesh=mesh,
        compiler_params=pltpu.CompilerParams(needs_layout_passes=False),
        scratch_types=[
            pltpu.VMEM((HD * PSTRIDE,), jnp.int32),
            pltpu.VMEM((P, QC), jnp.int32),
            pltpu.VMEM((P, QC), jnp.int32),
            pltpu.VMEM((HD, QC), jnp.float32),
        ],
    )
    def sc_kernel(t_hbm, c_hbm, j_hbm, out_hbm, ptab, cbuf, jbuf, obuf):
        cid = lax.axis_index("c")
        sid = lax.axis_index("s")
        wid = sid * NC + cid
        bh = wid % (BS * H)
        half = wid // (BS * H)
        b = bh // H
        h = bh % H

        pltpu.sync_copy(t_hbm.at[b, h], ptab)

        def do_chunk(qs):
            pltpu.sync_copy(c_hbm.at[b, h, :, pl.ds(qs, QC)], cbuf)
            pltpu.sync_copy(j_hbm.at[b, h, :, pl.ds(qs, QC)], jbuf)

            def group_body(g, _):
                qoff = g * LG

                init = tuple(jnp.zeros((2 * LG,), jnp.bfloat16)
                             for _ in range(DB))

                @plsc.parallel_loop(0, P, carry=init, unroll=UNROLL)
                def accs(p, accs):
                    jv = jbuf[p, pl.ds(qoff, LG)]
                    cv = plsc.bitcast(cbuf[p, pl.ds(qoff, LG)], jnp.bfloat16)
                    new = []
                    for j in range(DB):
                        tab = ptab.at[pl.ds(j * PSTRIDE, PSTRIDE)]
                        gv = plsc.load_gather(tab, [jv])
                        gb = plsc.bitcast(gv, jnp.bfloat16)
                        new.append(accs[j] + gb * cv)
                    return tuple(new)

                for j in range(DB):
                    lo, hi = plsc.unpack(accs[j],
                                         format=plsc.PackFormat.INTERLEAVED)
                    obuf[j, pl.ds(qoff, LG)] = lo + hi
                return 0

            lax.fori_loop(0, QC // LG, group_body, 0)
            pltpu.sync_copy(obuf, out_hbm.at[b, h, :, pl.ds(qs, QC)])

        q0 = half * (NQ // 2)
        for c in range(NQ // 2 // QC):
            do_chunk(q0 + c * QC)

    return sc_kernel(pair_tab, cpk, jvi)


def _stage3_body(ao_ref, q_ref, wo_ref, bo_ref, g_ref, beta_ref, out_ref):
    a = ao_ref[0].reshape(D, BQ).T
    o = jnp.dot(a, wo_ref[...].T, preferred_element_type=jnp.float32)
    o = o + bo_ref[...] + q_ref[0]
    mu = jnp.mean(o, axis=-1, keepdims=True)
    var = jnp.mean((o - mu) ** 2, axis=-1, keepdims=True)
    out_ref[0] = (o - mu) / jnp.sqrt(var + 1e-5) * g_ref[...] + beta_ref[...]


def _stage3(attn_t, query, W_out, b_out, ln_g, ln_b):
    grid = (BS, NQ // BQ)
    blk_rows = pl.BlockSpec((1, BQ, D), lambda b, i: (b, i, 0))
    blk_a = pl.BlockSpec((1, H, HD, BQ), lambda b, i: (b, 0, 0, i))
    blk_full = pl.BlockSpec((D, D), lambda b, i: (0, 0))
    blk_vec = pl.BlockSpec((D,), lambda b, i: (0,))
    return pl.pallas_call(
        _stage3_body,
        grid=grid,
        in_specs=[blk_a, blk_rows, blk_full, blk_vec, blk_vec, blk_vec],
        out_specs=blk_rows,
        out_shape=jax.ShapeDtypeStruct((BS, NQ, D), jnp.float32),
    )(attn_t, query, W_out, b_out, ln_g, ln_b)


@jax.jit
def _run(query, value, reference_points, W_off, b_off, W_sw, b_sw, W_v, b_v,
         W_out, b_out, ln_g, ln_b, spatial_shapes):
    ref2 = reference_points.reshape(BS, NQ, 1)
    v_proj, cpk, jvi = _stage1(query, value, ref2, W_v, b_v,
                               W_off, b_off, W_sw, b_sw)
    pair_tab = _pack_table(v_proj).reshape(BS, H, HD * PSTRIDE)
    attn_t = _sc_attend(pair_tab, cpk, jvi)
    return _stage3(attn_t, query, W_out, b_out, ln_g, ln_b)


def kernel(query, value, reference_points, W_off, b_off, W_sw, b_sw, W_v, b_v,
           W_out, b_out, ln_g, ln_b, spatial_shapes):
    return _run(query, value, reference_points, W_off, b_off, W_sw, b_sw,
                W_v, b_v, W_out, b_out, ln_g, ln_b, spatial_shapes)


# speedup vs baseline: 1.1843x; 1.0206x over previous
import functools

import jax
import jax.numpy as jnp
from jax import lax
from jax.experimental import pallas as pl
from jax.experimental.pallas import tpu as pltpu
from jax.experimental.pallas import tpu_sc as plsc

BS = 2
NQ = 2048
NV = 2048
D = 256
H = 8
P = 32
HD = D // H

BQ = 512
NC, NS = 2, 16
NW = NC * NS
QPW = (BS * NQ * H) // (NW * H) * 1
QC = 512
LG = 16
DB = 32
UNROLL = 2


def _pack_pair_words(lo, hi):
    lob = lo.astype(jnp.bfloat16).astype(jnp.float32)
    hib = hi.astype(jnp.bfloat16).astype(jnp.float32)
    lou = lax.bitcast_convert_type(lob, jnp.uint32) >> 16
    hiu = lax.bitcast_convert_type(hib, jnp.uint32) & jnp.uint32(0xFFFF0000)
    return lax.bitcast_convert_type(lou | hiu, jnp.int32)


def _stage1_body(q_ref, v_ref, r_ref, wv_ref, bv_ref, wo_ref, bo_ref,
                 wsw_ref, bsw_ref, vout_ref, c_ref, j_ref):
    q = q_ref[0]
    v = v_ref[0]
    vp = jnp.dot(v, wv_ref[...].T, preferred_element_type=jnp.float32) + bv_ref[...]
    vout_ref[0] = vp.T.reshape(H, HD, BQ)

    off = jnp.dot(q, wo_ref[...].T, preferred_element_type=jnp.float32) + bo_ref[...]
    r = r_ref[0]
    x = r * float(NV) - 1.0 + off

    sw = jnp.dot(q, wsw_ref[...].T, preferred_element_type=jnp.float32) + bsw_ref[...]
    sw3 = sw.reshape(BQ, H, P)
    m = jnp.max(sw3, axis=-1, keepdims=True)
    e = jnp.exp(sw3 - m)
    s = jnp.sum(e, axis=-1, keepdims=True)
    w = (e / s).reshape(BQ, H * P)

    xl = jnp.floor(x)
    lx = x - xl
    hx = 1.0 - lx
    a = jnp.where((xl >= 0.0) & (xl <= float(NV - 2)), w * hx, 0.0)
    bb = (jnp.where((xl >= -1.0) & (xl <= float(NV - 2)), w * lx, 0.0)
          + jnp.where(xl == float(NV - 1), w * hx, 0.0))
    jv = jnp.clip(xl + 1.0, 0.0, float(NV - 1)).astype(jnp.int32)
    cw = _pack_pair_words(a, bb)
    c_ref[0] = cw.T.reshape(H, P, BQ)
    j_ref[0] = jv.T.reshape(H, P, BQ)


def _stage1(query, value, ref_pts, W_v, b_v, W_off, b_off, W_sw, b_sw):
    grid = (BS, NQ // BQ)
    blk_rows = pl.BlockSpec((1, BQ, D), lambda b, i: (b, i, 0))
    blk_full = pl.BlockSpec((D, D), lambda b, i: (0, 0))
    blk_vec = pl.BlockSpec((D,), lambda b, i: (0,))
    blk_r = pl.BlockSpec((1, BQ, 1), lambda b, i: (b, i, 0))
    blk_t = pl.BlockSpec((1, H, P, BQ), lambda b, i: (b, 0, 0, i))
    blk_vt = pl.BlockSpec((1, H, HD, BQ), lambda b, i: (b, 0, 0, i))
    return pl.pallas_call(
        _stage1_body,
        grid=grid,
        in_specs=[blk_rows, blk_rows, blk_r, blk_full, blk_vec, blk_full,
                  blk_vec, blk_full, blk_vec],
        out_specs=[blk_vt, blk_t, blk_t],
        out_shape=[
            jax.ShapeDtypeStruct((BS, H, HD, NV), jnp.float32),
            jax.ShapeDtypeStruct((BS, H, P, NQ), jnp.int32),
            jax.ShapeDtypeStruct((BS, H, P, NQ), jnp.int32),
        ],
    )(query, value, ref_pts, W_v, b_v, W_off, b_off, W_sw, b_sw)


def _pack_body(v_ref, t_ref):
    v = v_ref[0, 0]
    z = jnp.zeros((HD, PSTRIDE - NV), jnp.float32)
    lo = jnp.concatenate([v[:, :1], v[:, :NV - 1], z], axis=1)
    hi = jnp.concatenate([v, z], axis=1)
    t_ref[0, 0] = _pack_pair_words(lo, hi)


def _pack_table(v_proj):
    return pl.pallas_call(
        _pack_body,
        grid=(BS, H),
        in_specs=[pl.BlockSpec((1, 1, HD, NV), lambda b, h: (b, h, 0, 0))],
        out_specs=pl.BlockSpec((1, 1, HD, PSTRIDE), lambda b, h: (b, h, 0, 0)),
        out_shape=jax.ShapeDtypeStruct((BS, H, HD, PSTRIDE), jnp.int32),
    )(v_proj)


PSTRIDE = 2064


def _sc_attend(pair_tab, cpk, jvi):
    mesh = plsc.VectorSubcoreMesh(core_axis_name="c", subcore_axis_name="s",
                                  num_cores=NC, num_subcores=NS)

    @functools.partial(
        pl.kernel,
        out_type=jax.ShapeDtypeStruct((BS, H, HD, NQ), jnp.float32),
        mesh=mesh,
        compiler_params=pltpu.CompilerParams(needs_layout_passes=False),
        scratch_types=[
            pltpu.VMEM((HD * PSTRIDE,), jnp.int32),
            pltpu.VMEM((P, QC), jnp.int32),
            pltpu.VMEM((P, QC), jnp.int32),
            pltpu.VMEM((HD, QC), jnp.float32),
        ],
    )
    def sc_kernel(t_hbm, c_hbm, j_hbm, out_hbm, ptab, cbuf, jbuf, obuf):
        cid = lax.axis_index("c")
        sid = lax.axis_index("s")
        wid = sid * NC + cid
        bh = wid % (BS * H)
        half = wid // (BS * H)
        b = bh // H
        h = bh % H

        pltpu.sync_copy(t_hbm.at[b, h], ptab)

        def do_chunk(qs):
            pltpu.sync_copy(c_hbm.at[b, h, :, pl.ds(qs, QC)], cbuf)
            pltpu.sync_copy(j_hbm.at[b, h, :, pl.ds(qs, QC)], jbuf)

            def group_body(g, _):
                qoff = g * LG

                init = tuple(jnp.zeros((2 * LG,), jnp.bfloat16)
                             for _ in range(DB))

                @plsc.parallel_loop(0, P, carry=init, unroll=UNROLL)
                def accs(p, accs):
                    jv = jbuf[p, pl.ds(qoff, LG)]
                    cv = plsc.bitcast(cbuf[p, pl.ds(qoff, LG)], jnp.bfloat16)
                    new = []
                    for j in range(DB):
                        tab = ptab.at[pl.ds(j * PSTRIDE, PSTRIDE)]
                        gv = plsc.load_gather(tab, [jv])
                        gb = plsc.bitcast(gv, jnp.bfloat16)
                        new.append(accs[j] + gb * cv)
                    return tuple(new)

                for j in range(DB):
                    lo, hi = plsc.unpack(accs[j],
                                         format=plsc.PackFormat.INTERLEAVED)
                    obuf[j, pl.ds(qoff, LG)] = lo + hi
                return 0

            lax.fori_loop(0, QC // LG, group_body, 0)
            pltpu.sync_copy(obuf, out_hbm.at[b, h, :, pl.ds(qs, QC)])

        q0 = half * (NQ // 2)
        for c in range(NQ // 2 // QC):
            do_chunk(q0 + c * QC)

    return sc_kernel(pair_tab, cpk, jvi)


def _stage3_body(ao_ref, q_ref, wo_ref, bo_ref, g_ref, beta_ref, out_ref):
    a = ao_ref[0].reshape(D, BQ).T
    o = jnp.dot(a, wo_ref[...].T, preferred_element_type=jnp.float32)
    o = o + bo_ref[...] + q_ref[0]
    mu = jnp.mean(o, axis=-1, keepdims=True)
    var = jnp.mean((o - mu) ** 2, axis=-1, keepdims=True)
    out_ref[0] = (o - mu) / jnp.sqrt(var + 1e-5) * g_ref[...] + beta_ref[...]


def _stage3(attn_t, query, W_out, b_out, ln_g, ln_b):
    grid = (BS, NQ // BQ)
    blk_rows = pl.BlockSpec((1, BQ, D), lambda b, i: (b, i, 0))
    blk_a = pl.BlockSpec((1, H, HD, BQ), lambda b, i: (b, 0, 0, i))
    blk_full = pl.BlockSpec((D, D), lambda b, i: (0, 0))
    blk_vec = pl.BlockSpec((D,), lambda b, i: (0,))
    return pl.pallas_call(
        _stage3_body,
        grid=grid,
        in_specs=[blk_a, blk_rows, blk_full, blk_vec, blk_vec, blk_vec],
        out_specs=blk_rows,
        out_shape=jax.ShapeDtypeStruct((BS, NQ, D), jnp.float32),
    )(attn_t, query, W_out, b_out, ln_g, ln_b)


@jax.jit
def _run(query, value, reference_points, W_off, b_off, W_sw, b_sw, W_v, b_v,
         W_out, b_out, ln_g, ln_b, spatial_shapes):
    ref2 = reference_points.reshape(BS, NQ, 1)
    v_proj, cpk, jvi = _stage1(query, value, ref2, W_v, b_v,
                               W_off, b_off, W_sw, b_sw)
    pair_tab = _pack_table(v_proj).reshape(BS, H, HD * PSTRIDE)
    attn_t = _sc_attend(pair_tab, cpk, jvi)
    return _stage3(attn_t, query, W_out, b_out, ln_g, ln_b)


def kernel(query, value, reference_points, W_off, b_off, W_sw, b_sw, W_v, b_v,
           W_out, b_out, ln_g, ln_b, spatial_shapes):
    return _run(query, value, reference_points, W_off, b_off, W_sw, b_sw,
                W_v, b_v, W_out, b_out, ln_g, ln_b, spatial_shapes)
